# Initial kernel scaffold; baseline (speedup 1.0000x reference)
import jax
import jax.numpy as jnp
from jax.experimental import pallas as pl


def kernel(feat, rgbs):
    raise NotImplementedError("write your pallas kernel here")


# speedup vs baseline: 4.1455x; 4.1455x over previous
import functools

import jax
import jax.numpy as jnp
from jax import lax
from jax.experimental import pallas as pl
from jax.experimental.pallas import tpu as pltpu
from jax.experimental.pallas import tpu_sc as plsc

N = 320000
D = 128
NSEG = 512
NC = 2
NS = 16
NW = NC * NS
NP = N // NW
CHUNK = 400
NK = NP // CHUNK
SUB = 80
NSUB = CHUNK // SUB
GROWS = N // SUB
L = 16

_mesh = plsc.VectorSubcoreMesh(core_axis_name="c", subcore_axis_name="s")


def _k1_body(feat_hbm, r_hbm, g_hbm, b_hbm,
             sums_out, cnt_out,
             shared_acc,
             r_v, g_v, b_v, gid_v, feat_v, cnt_v, stripe_v):
    cid = lax.axis_index("c")
    sid = lax.axis_index("s")
    wid = sid * NC + cid
    base = wid * NP

    zeros16 = jnp.zeros((L,), jnp.float32)
    ones16 = jnp.ones((L,), jnp.float32)
    for r in range(NSEG // NS):
        for q in range(D // L):
            stripe_v[jnp.int32(r), pl.ds(q * L, L)] = zeros16
    for r in range(NSEG // L):
        cnt_v[jnp.int32(r), :] = zeros16
    row0 = sid * (NSEG // NS)
    pltpu.sync_copy(stripe_v, shared_acc.at[pl.ds(row0, NSEG // NS), :])
    plsc.subcore_barrier()

    def step(k, carry):
        off = base + k * CHUNK
        pltpu.sync_copy(r_hbm.at[pl.ds(off, CHUNK)], r_v)
        pltpu.sync_copy(g_hbm.at[pl.ds(off, CHUNK)], g_v)
        pltpu.sync_copy(b_hbm.at[pl.ds(off, CHUNK)], b_v)
        pltpu.sync_copy(feat_hbm.at[pl.ds(off, CHUNK), :], feat_v)
        for m in range(CHUNK // L):
            r16 = r_v[pl.ds(m * L, L)]
            g16 = g_v[pl.ds(m * L, L)]
            b16 = b_v[pl.ds(m * L, L)]
            gid16 = r16 * 64 + g16 * 8 + b16
            gid_v[jnp.int32(m // (SUB // L)), pl.ds((m % (SUB // L)) * L, L)] = gid16
            plsc.addupdate_scatter(
                cnt_v, [gid16 >> 4, gid16 & 15], ones16)
        for j in range(NSUB):
            pltpu.sync_copy(feat_v.at[pl.ds(j * SUB, SUB), :],
                            shared_acc.at[gid_v.at[jnp.int32(j)]], add=True)
        return carry

    lax.fori_loop(jnp.int32(0), jnp.int32(NK), step, jnp.int32(0))
    plsc.subcore_barrier()

    pltpu.sync_copy(shared_acc.at[pl.ds(row0, NSEG // NS), :], stripe_v)
    pltpu.sync_copy(stripe_v, sums_out.at[cid, pl.ds(row0, NSEG // NS), :])
    pltpu.sync_copy(cnt_v, cnt_out.at[wid])


_k1 = pl.kernel(
    _k1_body,
    out_type=(
        jax.ShapeDtypeStruct((NC, NSEG, D), jnp.float32),
        jax.ShapeDtypeStruct((NW, NSEG // L, L), jnp.float32),
    ),
    mesh=_mesh,
    compiler_params=pltpu.CompilerParams(needs_layout_passes=False),
    scratch_types=[
        pltpu.VMEM_SHARED((NSEG, D), jnp.float32),
        pltpu.VMEM((CHUNK,), jnp.int32),
        pltpu.VMEM((CHUNK,), jnp.int32),
        pltpu.VMEM((CHUNK,), jnp.int32),
        pltpu.VMEM((NSUB, SUB), jnp.int32),
        pltpu.VMEM((CHUNK, D), jnp.float32),
        pltpu.VMEM((NSEG // L, L), jnp.float32),
        pltpu.VMEM((NSEG // NS, D), jnp.float32),
    ],
)


def _combine_body(sp_ref, cp_ref, o_ref):
    s = sp_ref[0] + sp_ref[1]
    c = jnp.sum(cp_ref[...], axis=0)
    o_ref[...] = s / c[:, None]


def _k3_body(pooled_hbm, r_hbm, g_hbm, b_hbm, out_hbm,
             r_v, g_v, b_v, gid_v, rows_v, sem):
    cid = lax.axis_index("c")
    sid = lax.axis_index("s")
    wid = sid * NC + cid
    base = wid * NP

    def step(k, carry):
        off = base + k * CHUNK
        pltpu.sync_copy(r_hbm.at[pl.ds(off, CHUNK)], r_v)
        pltpu.sync_copy(g_hbm.at[pl.ds(off, CHUNK)], g_v)
        pltpu.sync_copy(b_hbm.at[pl.ds(off, CHUNK)], b_v)
        for m in range(CHUNK // L):
            r16 = r_v[pl.ds(m * L, L)]
            g16 = g_v[pl.ds(m * L, L)]
            b16 = b_v[pl.ds(m * L, L)]
            gid16 = r16 * 64 + g16 * 8 + b16
            gid_v[jnp.int32(m // (SUB // L)), pl.ds((m % (SUB // L)) * L, L)] = gid16
        for j in range(NSUB):
            pltpu.async_copy(pooled_hbm.at[gid_v.at[jnp.int32(j)]],
                             rows_v.at[pl.ds(j * SUB, SUB), :], sem).wait()
        pltpu.sync_copy(rows_v, out_hbm.at[pl.ds(base + k * CHUNK, CHUNK), :])
        return carry

    lax.fori_loop(jnp.int32(0), jnp.int32(NK), step, jnp.int32(0))


_k3 = pl.kernel(
    _k3_body,
    out_type=jax.ShapeDtypeStruct((N, D), jnp.float32),
    mesh=_mesh,
    compiler_params=pltpu.CompilerParams(needs_layout_passes=False),
    scratch_types=[
        pltpu.VMEM((CHUNK,), jnp.int32),
        pltpu.VMEM((CHUNK,), jnp.int32),
        pltpu.VMEM((CHUNK,), jnp.int32),
        pltpu.VMEM((NSUB, SUB), jnp.int32),
        pltpu.VMEM((CHUNK, D), jnp.float32),
        pltpu.SemaphoreType.DMA,
    ],
)


def kernel(feat, rgbs):
    rgbs32 = rgbs.astype(jnp.int32)
    r = rgbs32[:, 0]
    g = rgbs32[:, 1]
    b = rgbs32[:, 2]
    sums, cnts = _k1(feat, r, g, b)
    pooled = pl.pallas_call(
        _combine_body,
        out_shape=jax.ShapeDtypeStruct((NSEG, D), jnp.float32),
    )(sums, cnts.reshape(NW, NSEG))
    return _k3(pooled, r, g, b)


# speedup vs baseline: 10.0531x; 2.4250x over previous
import jax
import jax.numpy as jnp
from jax import lax
from jax.experimental import pallas as pl
from jax.experimental.pallas import tpu as pltpu
from jax.experimental.pallas import tpu_sc as plsc

N = 320000
D = 128
NSEG = 512
NC = 2
NS = 16
NW = NC * NS
NP = N // NW
CHUNK = 400
NK = NP // CHUNK
SUB = 80
NSUB = CHUNK // SUB
L = 16
RPT = NSEG // NS

_mesh = plsc.VectorSubcoreMesh(core_axis_name="c", subcore_axis_name="s")
_params = pltpu.CompilerParams(needs_layout_passes=False)


def _compute_gid(r_v, g_v, b_v, gid_v):
    for m in range(CHUNK // L):
        r16 = r_v[pl.ds(m * L, L)]
        g16 = g_v[pl.ds(m * L, L)]
        b16 = b_v[pl.ds(m * L, L)]
        gid16 = r16 * 64 + g16 * 8 + b16
        gid_v[jnp.int32(m // (SUB // L)), pl.ds((m % (SUB // L)) * L, L)] = gid16


def _k1_body(feat_hbm, r_hbm, g_hbm, b_hbm,
             sums_out, cnt_out,
             shared_acc,
             r_a, g_a, b_a, f_a, r_b, g_b, b_b, f_b,
             gid_v, cnt_v, stripe_v, sem_a, sem_b, sem_s):
    cid = lax.axis_index("c")
    sid = lax.axis_index("s")
    wid = sid * NC + cid
    base = wid * NP

    zeros16 = jnp.zeros((L,), jnp.float32)
    ones16 = jnp.ones((L,), jnp.float32)
    for r in range(RPT):
        for q in range(D // L):
            stripe_v[jnp.int32(r), pl.ds(q * L, L)] = zeros16
    for r in range(NSEG // L):
        cnt_v[jnp.int32(r), :] = zeros16
    row0 = sid * RPT
    pltpu.sync_copy(stripe_v, shared_acc.at[pl.ds(row0, RPT), :])
    plsc.subcore_barrier()

    def fire_load(k, rv, gv, bv, fv, sem):
        off = base + k * jnp.int32(CHUNK)
        pltpu.async_copy(r_hbm.at[pl.ds(off, CHUNK)], rv, sem)
        pltpu.async_copy(g_hbm.at[pl.ds(off, CHUNK)], gv, sem)
        pltpu.async_copy(b_hbm.at[pl.ds(off, CHUNK)], bv, sem)
        pltpu.async_copy(feat_hbm.at[pl.ds(off, CHUNK), :], fv, sem)

    def wait_load(rv, gv, bv, fv, sem):
        pltpu.make_async_copy(r_hbm.at[pl.ds(base, CHUNK)], rv, sem).wait()
        pltpu.make_async_copy(g_hbm.at[pl.ds(base, CHUNK)], gv, sem).wait()
        pltpu.make_async_copy(b_hbm.at[pl.ds(base, CHUNK)], bv, sem).wait()
        pltpu.make_async_copy(feat_hbm.at[pl.ds(base, CHUNK), :], fv,
                              sem).wait()

    def process(rv, gv, bv, fv):
        _compute_gid(rv, gv, bv, gid_v)
        for j in range(NSUB):
            pltpu.async_copy(fv.at[pl.ds(j * SUB, SUB), :],
                             shared_acc.at[gid_v.at[jnp.int32(j)]],
                             sem_s, add=True)
        for m in range(CHUNK // L):
            gid16 = gid_v[jnp.int32(m // (SUB // L)),
                          pl.ds((m % (SUB // L)) * L, L)]
            plsc.addupdate_scatter(cnt_v, [gid16 >> 4, gid16 & 15], ones16)
        for j in range(NSUB):
            pltpu.make_async_copy(fv.at[pl.ds(j * SUB, SUB), :],
                                  shared_acc.at[gid_v.at[jnp.int32(j)]],
                                  sem_s).wait()

    fire_load(jnp.int32(0), r_a, g_a, b_a, f_a, sem_a)

    def pair(i, carry):
        k1 = 2 * i + 1
        wait_load(r_a, g_a, b_a, f_a, sem_a)
        fire_load(k1, r_b, g_b, b_b, f_b, sem_b)
        process(r_a, g_a, b_a, f_a)
        wait_load(r_b, g_b, b_b, f_b, sem_b)
        fire_load(k1 + 1, r_a, g_a, b_a, f_a, sem_a)
        process(r_b, g_b, b_b, f_b)
        return carry

    lax.fori_loop(jnp.int32(0), jnp.int32((NK - 1) // 2), pair, jnp.int32(0))
    wait_load(r_a, g_a, b_a, f_a, sem_a)
    process(r_a, g_a, b_a, f_a)
    plsc.subcore_barrier()

    pltpu.sync_copy(shared_acc.at[pl.ds(row0, RPT), :], stripe_v)
    pltpu.sync_copy(stripe_v, sums_out.at[cid, pl.ds(row0, RPT), :])
    pltpu.sync_copy(cnt_v, cnt_out.at[wid])


_k1 = pl.kernel(
    _k1_body,
    out_type=(
        jax.ShapeDtypeStruct((NC, NSEG, D), jnp.float32),
        jax.ShapeDtypeStruct((NW, NSEG // L, L), jnp.float32),
    ),
    mesh=_mesh,
    compiler_params=_params,
    scratch_types=[
        pltpu.VMEM_SHARED((NSEG, D), jnp.float32),
        pltpu.VMEM((CHUNK,), jnp.int32),
        pltpu.VMEM((CHUNK,), jnp.int32),
        pltpu.VMEM((CHUNK,), jnp.int32),
        pltpu.VMEM((CHUNK, D), jnp.float32),
        pltpu.VMEM((CHUNK,), jnp.int32),
        pltpu.VMEM((CHUNK,), jnp.int32),
        pltpu.VMEM((CHUNK,), jnp.int32),
        pltpu.VMEM((CHUNK, D), jnp.float32),
        pltpu.VMEM((NSUB, SUB), jnp.int32),
        pltpu.VMEM((NSEG // L, L), jnp.float32),
        pltpu.VMEM((RPT, D), jnp.float32),
        pltpu.SemaphoreType.DMA,
        pltpu.SemaphoreType.DMA,
        pltpu.SemaphoreType.DMA,
    ],
)


def _combine_body(sp_ref, cp_ref, o_ref):
    s = sp_ref[0] + sp_ref[1]
    c = jnp.sum(cp_ref[...], axis=0)
    o_ref[...] = s / c[:, None]


def _k3_body(pooled_hbm, r_hbm, g_hbm, b_hbm, out_hbm,
             shared_pool,
             r_a, g_a, b_a, r_b, g_b, b_b,
             gid_a, gid_b, rows_a, rows_b, stage_v,
             sem_la, sem_lb, sem_ga, sem_gb, sem_sa, sem_sb):
    cid = lax.axis_index("c")
    sid = lax.axis_index("s")
    wid = sid * NC + cid
    base = wid * NP
    row0 = sid * RPT

    pltpu.sync_copy(pooled_hbm.at[pl.ds(row0, RPT), :], stage_v)
    pltpu.sync_copy(stage_v, shared_pool.at[pl.ds(row0, RPT), :])
    plsc.subcore_barrier()

    def fire_rgb(k, rv, gv, bv, sem):
        off = base + k * jnp.int32(CHUNK)
        pltpu.async_copy(r_hbm.at[pl.ds(off, CHUNK)], rv, sem)
        pltpu.async_copy(g_hbm.at[pl.ds(off, CHUNK)], gv, sem)
        pltpu.async_copy(b_hbm.at[pl.ds(off, CHUNK)], bv, sem)

    def wait_rgb(rv, gv, bv, sem):
        pltpu.make_async_copy(r_hbm.at[pl.ds(base, CHUNK)], rv, sem).wait()
        pltpu.make_async_copy(g_hbm.at[pl.ds(base, CHUNK)], gv, sem).wait()
        pltpu.make_async_copy(b_hbm.at[pl.ds(base, CHUNK)], bv, sem).wait()

    def fire_gath(gid_v, rows_v, sem):
        for j in range(NSUB):
            pltpu.async_copy(shared_pool.at[gid_v.at[jnp.int32(j)]],
                             rows_v.at[pl.ds(j * SUB, SUB), :], sem)

    def drain_gath(gid_v, rows_v, sem):
        for j in range(NSUB):
            pltpu.make_async_copy(shared_pool.at[gid_v.at[jnp.int32(j)]],
                                  rows_v.at[pl.ds(j * SUB, SUB), :],
                                  sem).wait()

    def fire_store(k, rows_v, sem):
        off = base + k * jnp.int32(CHUNK)
        pltpu.async_copy(rows_v, out_hbm.at[pl.ds(off, CHUNK), :], sem)

    def drain_store(rows_v, sem):
        pltpu.make_async_copy(rows_v, out_hbm.at[pl.ds(base, CHUNK), :],
                              sem).wait()

    def head(k, rv, gv, bv, gid_v, rows_v, sem_l, sem_g, sem_s, next_k):
        wait_rgb(rv, gv, bv, sem_l)
        _compute_gid(rv, gv, bv, gid_v)
        fire_rgb(next_k, rv, gv, bv, sem_l)
        fire_gath(gid_v, rows_v, sem_g)
        drain_gath(gid_v, rows_v, sem_g)
        fire_store(k, rows_v, sem_s)

    fire_rgb(jnp.int32(0), r_a, g_a, b_a, sem_la)
    fire_rgb(jnp.int32(1), r_b, g_b, b_b, sem_lb)
    head(jnp.int32(0), r_a, g_a, b_a, gid_a, rows_a, sem_la, sem_ga, sem_sa,
         jnp.int32(2))
    head(jnp.int32(1), r_b, g_b, b_b, gid_b, rows_b, sem_lb, sem_gb, sem_sb,
         jnp.int32(3))

    def pair(i, carry):
        ka = 2 * i + 2
        kb = 2 * i + 3
        wait_rgb(r_a, g_a, b_a, sem_la)
        _compute_gid(r_a, g_a, b_a, gid_a)
        fire_rgb(jnp.minimum(ka + 2, jnp.int32(NK - 1)), r_a, g_a, b_a,
                 sem_la)
        drain_store(rows_a, sem_sa)
        fire_gath(gid_a, rows_a, sem_ga)
        drain_gath(gid_a, rows_a, sem_ga)
        fire_store(ka, rows_a, sem_sa)
        wait_rgb(r_b, g_b, b_b, sem_lb)
        _compute_gid(r_b, g_b, b_b, gid_b)
        fire_rgb(jnp.minimum(kb + 2, jnp.int32(NK - 1)), r_b, g_b, b_b,
                 sem_lb)
        drain_store(rows_b, sem_sb)
        fire_gath(gid_b, rows_b, sem_gb)
        drain_gath(gid_b, rows_b, sem_gb)
        fire_store(kb, rows_b, sem_sb)
        return carry

    lax.fori_loop(jnp.int32(0), jnp.int32((NK - 3) // 2), pair, jnp.int32(0))
    wait_rgb(r_a, g_a, b_a, sem_la)
    _compute_gid(r_a, g_a, b_a, gid_a)
    drain_store(rows_a, sem_sa)
    fire_gath(gid_a, rows_a, sem_ga)
    drain_gath(gid_a, rows_a, sem_ga)
    fire_store(jnp.int32(NK - 1), rows_a, sem_sa)
    wait_rgb(r_b, g_b, b_b, sem_lb)
    drain_store(rows_b, sem_sb)
    drain_store(rows_a, sem_sa)


_k3 = pl.kernel(
    _k3_body,
    out_type=jax.ShapeDtypeStruct((N, D), jnp.float32),
    mesh=_mesh,
    compiler_params=_params,
    scratch_types=[
        pltpu.VMEM_SHARED((NSEG, D), jnp.float32),
        pltpu.VMEM((CHUNK,), jnp.int32),
        pltpu.VMEM((CHUNK,), jnp.int32),
        pltpu.VMEM((CHUNK,), jnp.int32),
        pltpu.VMEM((CHUNK,), jnp.int32),
        pltpu.VMEM((CHUNK,), jnp.int32),
        pltpu.VMEM((CHUNK,), jnp.int32),
        pltpu.VMEM((NSUB, SUB), jnp.int32),
        pltpu.VMEM((NSUB, SUB), jnp.int32),
        pltpu.VMEM((CHUNK, D), jnp.float32),
        pltpu.VMEM((CHUNK, D), jnp.float32),
        pltpu.VMEM((RPT, D), jnp.float32),
        pltpu.SemaphoreType.DMA,
        pltpu.SemaphoreType.DMA,
        pltpu.SemaphoreType.DMA,
        pltpu.SemaphoreType.DMA,
        pltpu.SemaphoreType.DMA,
        pltpu.SemaphoreType.DMA,
    ],
)


def kernel(feat, rgbs):
    rgbs32 = rgbs.astype(jnp.int32)
    r = rgbs32[:, 0]
    g = rgbs32[:, 1]
    b = rgbs32[:, 2]
    sums, cnts = _k1(feat, r, g, b)
    pooled = pl.pallas_call(
        _combine_body,
        out_shape=jax.ShapeDtypeStruct((NSEG, D), jnp.float32),
    )(sums, cnts.reshape(NW, NSEG))
    return _k3(pooled, r, g, b)
